# baseline (speedup 1.0000x reference)
import dataclasses
import functools

import jax
import jax.numpy as jnp
from jax import lax
from jax.experimental import pallas as pl
from jax.experimental.pallas import tpu as pltpu
from jax.experimental.pallas import tpu_sc as plsc

LANES = 16
NBUF = 4
QBLK = 32


def _sc_agg(feats2d, bcast2d, nbr, *, R, BD, K, TOP_K, NW):
    RW = R // NW
    CH = BD // LANES
    GROUP = 16 // TOP_K
    NG = RW // GROUP

    mesh = plsc.VectorSubcoreMesh(core_axis_name="c", subcore_axis_name="s")
    cp = pltpu.CompilerParams()
    if "needs_layout_passes" in pltpu.CompilerParams.__dataclass_fields__:
        cp = dataclasses.replace(cp, needs_layout_passes=False)

    @functools.partial(
        pl.kernel,
        compiler_params=cp,
        out_type=jax.ShapeDtypeStruct((R, BD), jnp.float32),
        mesh=mesh,
        scratch_types=[
            pltpu.VMEM((RW, K), jnp.int32),
            pltpu.VMEM((QBLK, BD), jnp.float32),
            pltpu.VMEM((NBUF, 16, BD), jnp.float32),
            pltpu.VMEM((NG, 16), jnp.int32),
            [pltpu.SemaphoreType.DMA] * NBUF,
            [pltpu.SemaphoreType.DMA] * 2,
            [pltpu.SemaphoreType.DMA] * 2,
        ],
    )
    def k(feats_hbm, bcast_hbm, nbr_hbm, agg_hbm,
          nbr_v, q_v, big_v, sel_v, nb_sems, g_sems, w_sems):
        wid = lax.axis_index("c") * 16 + lax.axis_index("s")
        base = wid * RW

        pltpu.sync_copy(nbr_hbm.at[pl.ds(base, RW)], nbr_v)

        def nb_gather(r, j):
            return pltpu.make_async_copy(
                feats_hbm.at[nbr_v.at[r]], big_v.at[j], nb_sems[j])

        lane = lax.iota(jnp.int32, LANES)
        zero = jnp.zeros((LANES,), jnp.float32)

        for j in range(NBUF):
            nb_gather(j, j).start()

        @pl.loop(0, RW, step=NBUF)
        def _(r):
            @pl.when(lax.rem(r, QBLK) == 0)
            def _():
                pltpu.sync_copy(
                    feats_hbm.at[pl.ds(pl.multiple_of(base + r, QBLK), QBLK)],
                    q_v)

            for j in range(NBUF):
                nb_gather(r + j, j).wait()

                def cstep(c, accs):
                    qc = q_v[lax.rem(r + j, QBLK), pl.ds(c * LANES, LANES)]
                    return tuple(
                        accs[t] + qc * big_v[j, t, pl.ds(c * LANES, LANES)]
                        for t in range(K)
                    )

                accs = lax.fori_loop(0, CH, cstep, (zero,) * K, unroll=2)

                sims = zero
                for t in range(K):
                    sims = jnp.where(lane == t, jnp.sum(accs[t]), sims)

                skeys, svals = plsc.sort_key_val(
                    sims, nbr_v[r + j], descending=True)
                del skeys
                rg = (r + j) // GROUP
                rows = jnp.full((LANES,), rg, jnp.int32)
                cols = lax.rem(r + j, GROUP) * TOP_K + lane
                plsc.store_scatter(sel_v, [rows, cols], svals,
                                   mask=lane < TOP_K)

                @pl.when(r + j + NBUF < RW)
                def _():
                    nb_gather(r + j + NBUF, j).start()

        def g_gather(g, j):
            return pltpu.make_async_copy(
                bcast_hbm.at[sel_v.at[g]], big_v.at[j], g_sems[j])

        def w_write(g, j):
            return pltpu.make_async_copy(
                big_v.at[2 + j, pl.ds(0, 2 * GROUP)],
                agg_hbm.at[pl.ds(pl.multiple_of(base + g * GROUP, 8),
                                 2 * GROUP)],
                w_sems[j])

        g_gather(0, 0).start()
        g_gather(1, 1).start()

        @pl.loop(0, NG, step=4)
        def _(g):
            @pl.when(g >= 4)
            def _():
                w_write(g - 4, 0).wait()
                w_write(g - 2, 1).wait()

            for jj in range(4):
                gb = jj % 2
                wb = jj // 2
                g_gather(g + jj, gb).wait()

                @pl.loop(0, BD, step=LANES)
                def _(c):
                    for i in range(GROUP):
                        row = (jj % 2) * GROUP + i
                        s01 = (big_v[gb, TOP_K * i + 0, pl.ds(c, LANES)]
                               + big_v[gb, TOP_K * i + 1, pl.ds(c, LANES)])
                        s23 = (big_v[gb, TOP_K * i + 2, pl.ds(c, LANES)]
                               + big_v[gb, TOP_K * i + 3, pl.ds(c, LANES)])
                        big_v[2 + wb, row, pl.ds(c, LANES)] = s01 + s23

                @pl.when(g + jj + 2 < NG)
                def _():
                    g_gather(g + jj + 2, gb).start()

                if jj % 2 == 1:
                    w_write(g + jj - 1, wb).start()

        w_write(NG - 4, 0).wait()
        w_write(NG - 2, 1).wait()

    return k(feats2d, bcast2d, nbr)


def _tc_flatten(x3d, wargs, *, R, B, D, BLK):
    def body(x_ref, wt_ref, b_ref, o_ref, p_ref):
        w1 = wt_ref[:D, :]
        for bb in range(B):
            xb = x_ref[:, bb, :]
            sl = pl.ds(bb * D, D)
            o_ref[:, sl] = xb
            p_ref[:, sl] = (
                jnp.dot(xb, w1, preferred_element_type=jnp.float32)
                + b_ref[...]
            )

    return pl.pallas_call(
        body,
        grid=(R // BLK,),
        in_specs=[
            pl.BlockSpec((BLK, B, D), lambda i: (i, 0, 0)),
            pl.BlockSpec((2 * D, D), lambda i: (0, 0)),
            pl.BlockSpec((1, D), lambda i: (0, 0)),
        ],
        out_specs=[
            pl.BlockSpec((BLK, B * D), lambda i: (i, 0)),
            pl.BlockSpec((BLK, B * D), lambda i: (i, 0)),
        ],
        out_shape=[
            jax.ShapeDtypeStruct((R, B * D), jnp.float32),
            jax.ShapeDtypeStruct((R, B * D), jnp.float32),
        ],
    )(x3d, *wargs)


def _tc_add_w2(p2d, a2d, Wt, *, R, B, D, TOP_K, BLK):
    BD = B * D

    def body(p_ref, a_ref, wt_ref, o_ref):
        w2 = wt_ref[D:, :]
        for bb in range(B):
            sl = pl.ds(bb * D, D)
            o_ref[:, bb, :] = (
                p_ref[:, sl]
                + jnp.dot(a_ref[:, sl] * (1.0 / TOP_K), w2,
                          preferred_element_type=jnp.float32)
            )

    return pl.pallas_call(
        body,
        grid=(R // BLK,),
        in_specs=[
            pl.BlockSpec((BLK, BD), lambda i: (i, 0)),
            pl.BlockSpec((BLK, BD), lambda i: (i, 0)),
            pl.BlockSpec((2 * D, D), lambda i: (0, 0)),
        ],
        out_specs=pl.BlockSpec((BLK, B, D), lambda i: (i, 0, 0)),
        out_shape=jax.ShapeDtypeStruct((R, B, D), jnp.float32),
    )(p2d, a2d, Wt)


def kernel(bcast_by_region, feats_by_region, neighbor_indices, W, b):
    R, B, D = bcast_by_region.shape
    K = neighbor_indices.shape[1]
    TOP_K = 4
    BD = B * D
    NW = 32

    feats2d = feats_by_region.reshape(R, BD)
    Wt = W.T
    bcast2d, part1 = _tc_flatten(bcast_by_region, (Wt, b.reshape(1, D)),
                                 R=R, B=B, D=D, BLK=512)

    agg2d = _sc_agg(feats2d, bcast2d, neighbor_indices,
                    R=R, BD=BD, K=K, TOP_K=TOP_K, NW=NW)

    return _tc_add_w2(part1, agg2d, Wt,
                      R=R, B=B, D=D, TOP_K=TOP_K, BLK=512)


# speedup vs baseline: 1.0251x; 1.0251x over previous
import dataclasses
import functools

import jax
import jax.numpy as jnp
from jax import lax
from jax.experimental import pallas as pl
from jax.experimental.pallas import tpu as pltpu
from jax.experimental.pallas import tpu_sc as plsc

LANES = 16
NBUF = 4
QBLK = 32


def _sc_agg(feats2d, bcast2d, nbr, *, R, BD, K, TOP_K, NW):
    RW = R // NW
    CH = BD // LANES
    GROUP = 16 // TOP_K
    NG = RW // GROUP

    mesh = plsc.VectorSubcoreMesh(core_axis_name="c", subcore_axis_name="s")
    cp = pltpu.CompilerParams()
    if "needs_layout_passes" in pltpu.CompilerParams.__dataclass_fields__:
        cp = dataclasses.replace(cp, needs_layout_passes=False)

    @functools.partial(
        pl.kernel,
        compiler_params=cp,
        out_type=jax.ShapeDtypeStruct((R, BD), jnp.float32),
        mesh=mesh,
        scratch_types=[
            pltpu.VMEM((RW, K), jnp.int32),
            pltpu.VMEM((QBLK, BD), jnp.float32),
            pltpu.VMEM((NBUF, 16, BD), jnp.float32),
            pltpu.VMEM((NG, 16), jnp.int32),
            [pltpu.SemaphoreType.DMA] * NBUF,
            [pltpu.SemaphoreType.DMA] * 2,
            [pltpu.SemaphoreType.DMA] * 2,
        ],
    )
    def k(feats_hbm, bcast_hbm, nbr_hbm, agg_hbm,
          nbr_v, q_v, big_v, sel_v, nb_sems, g_sems, w_sems):
        wid = lax.axis_index("c") * 16 + lax.axis_index("s")
        base = wid * RW

        pltpu.sync_copy(nbr_hbm.at[pl.ds(base, RW)], nbr_v)

        def nb_gather(r, j):
            return pltpu.make_async_copy(
                feats_hbm.at[nbr_v.at[r]], big_v.at[j], nb_sems[j])

        lane = lax.iota(jnp.int32, LANES)
        zero = jnp.zeros((LANES,), jnp.float32)

        for j in range(NBUF):
            nb_gather(j, j).start()

        @pl.loop(0, RW, step=NBUF)
        def _(r):
            @pl.when(lax.rem(r, QBLK) == 0)
            def _():
                pltpu.sync_copy(
                    feats_hbm.at[pl.ds(pl.multiple_of(base + r, QBLK), QBLK)],
                    q_v)

            for j in range(NBUF):
                nb_gather(r + j, j).wait()

                def cstep(c, accs):
                    qc = q_v[lax.rem(r + j, QBLK), pl.ds(c * LANES, LANES)]
                    return tuple(
                        accs[t] + qc * big_v[j, t, pl.ds(c * LANES, LANES)]
                        for t in range(K)
                    )

                accs = lax.fori_loop(0, CH, cstep, (zero,) * K, unroll=2)

                sims = zero
                for t in range(K):
                    sims = jnp.where(lane == t, jnp.sum(accs[t]), sims)

                skeys, svals = plsc.sort_key_val(
                    sims, nbr_v[r + j], descending=True)
                del skeys
                rg = (r + j) // GROUP
                rows = jnp.full((LANES,), rg, jnp.int32)
                cols = lax.rem(r + j, GROUP) * TOP_K + lane
                plsc.store_scatter(sel_v, [rows, cols], svals,
                                   mask=lane < TOP_K)

                @pl.when(r + j + NBUF < RW)
                def _():
                    nb_gather(r + j + NBUF, j).start()

        def g_gather(g, j):
            return pltpu.make_async_copy(
                bcast_hbm.at[sel_v.at[g]], big_v.at[j], g_sems[j])

        def w_write(g, j):
            return pltpu.make_async_copy(
                big_v.at[2 + j, pl.ds(0, 2 * GROUP)],
                agg_hbm.at[pl.ds(pl.multiple_of(base + g * GROUP, 8),
                                 2 * GROUP)],
                w_sems[j])

        g_gather(0, 0).start()
        g_gather(1, 1).start()

        @pl.loop(0, NG, step=4)
        def _(g):
            @pl.when(g >= 4)
            def _():
                w_write(g - 4, 0).wait()
                w_write(g - 2, 1).wait()

            for jj in range(4):
                gb = jj % 2
                wb = jj // 2
                g_gather(g + jj, gb).wait()

                @pl.loop(0, BD, step=LANES)
                def _(c):
                    for i in range(GROUP):
                        row = (jj % 2) * GROUP + i
                        s01 = (big_v[gb, TOP_K * i + 0, pl.ds(c, LANES)]
                               + big_v[gb, TOP_K * i + 1, pl.ds(c, LANES)])
                        s23 = (big_v[gb, TOP_K * i + 2, pl.ds(c, LANES)]
                               + big_v[gb, TOP_K * i + 3, pl.ds(c, LANES)])
                        big_v[2 + wb, row, pl.ds(c, LANES)] = s01 + s23

                @pl.when(g + jj + 2 < NG)
                def _():
                    g_gather(g + jj + 2, gb).start()

                if jj % 2 == 1:
                    w_write(g + jj - 1, wb).start()

        w_write(NG - 4, 0).wait()
        w_write(NG - 2, 1).wait()

    return k(feats2d, bcast2d, nbr)


def _tc_flatten(x3d, *, R, B, D, BLK):
    def body(x_ref, o_ref):
        for bb in range(B):
            o_ref[:, pl.ds(bb * D, D)] = x_ref[:, bb, :]

    return pl.pallas_call(
        body,
        grid=(R // BLK,),
        in_specs=[pl.BlockSpec((BLK, B, D), lambda i: (i, 0, 0))],
        out_specs=pl.BlockSpec((BLK, B * D), lambda i: (i, 0)),
        out_shape=jax.ShapeDtypeStruct((R, B * D), jnp.float32),
    )(x3d)


def _tc_linear(x2d, a2d, Wt, b2, *, R, B, D, TOP_K, BLK):
    BD = B * D

    def body(x_ref, a_ref, wt_ref, b_ref, o_ref):
        w1 = wt_ref[:D, :]
        w2 = wt_ref[D:, :]
        for bb in range(B):
            sl = pl.ds(bb * D, D)
            o_ref[:, bb, :] = (
                jnp.dot(x_ref[:, sl], w1, preferred_element_type=jnp.float32)
                + jnp.dot(a_ref[:, sl] * (1.0 / TOP_K), w2,
                          preferred_element_type=jnp.float32)
                + b_ref[...]
            )

    return pl.pallas_call(
        body,
        grid=(R // BLK,),
        in_specs=[
            pl.BlockSpec((BLK, BD), lambda i: (i, 0)),
            pl.BlockSpec((BLK, BD), lambda i: (i, 0)),
            pl.BlockSpec((2 * D, D), lambda i: (0, 0)),
            pl.BlockSpec((1, D), lambda i: (0, 0)),
        ],
        out_specs=pl.BlockSpec((BLK, B, D), lambda i: (i, 0, 0)),
        out_shape=jax.ShapeDtypeStruct((R, B, D), jnp.float32),
    )(x2d, a2d, Wt, b2)


def kernel(bcast_by_region, feats_by_region, neighbor_indices, W, b):
    R, B, D = bcast_by_region.shape
    K = neighbor_indices.shape[1]
    TOP_K = 4
    BD = B * D
    NW = 32

    feats2d = feats_by_region.reshape(R, BD)
    bcast2d = _tc_flatten(bcast_by_region, R=R, B=B, D=D, BLK=512)

    agg2d = _sc_agg(feats2d, bcast2d, neighbor_indices,
                    R=R, BD=BD, K=K, TOP_K=TOP_K, NW=NW)

    return _tc_linear(bcast2d, agg2d, W.T, b.reshape(1, D),
                      R=R, B=B, D=D, TOP_K=TOP_K, BLK=512)
